# baseline (speedup 1.0000x reference)
import functools

import jax
import jax.numpy as jnp
from jax import lax
from jax.experimental import pallas as pl
from jax.experimental.pallas import tpu as pltpu
from jax.experimental.pallas import tpu_sc as plsc

S = 10
H = 8
D = 128
LN_EPS = 1e-5

NUM_WORKERS = 32
CN = 8


def _oddeven_merge(lo, hi, r):
    step = r * 2
    if step < hi - lo:
        yield from _oddeven_merge(lo, hi, step)
        yield from _oddeven_merge(lo + r, hi, step)
        yield from ((i, i + r) for i in range(lo + r, hi - r, step))
    else:
        yield (lo, lo + r)


def _oddeven_sort(lo, hi):
    if (hi - lo) >= 1:
        mid = lo + ((hi - lo) // 2)
        yield from _oddeven_sort(lo, mid)
        yield from _oddeven_sort(mid + 1, hi)
        yield from _oddeven_merge(lo, hi + 1, 1)


def _network(n):
    p = 1
    while p < n:
        p *= 2
    return [(a, b) for (a, b) in _oddeven_sort(0, p - 1) if b < n]


_NET = _network(S)


def _sortnet_body(x_ref, idx8_ref, diff_ref):
    vals = [x_ref[i] for i in range(S)]
    bn = vals[0].shape[0]
    pws = [jnp.full((bn, D), 8 << i, jnp.int32) for i in range(S)]
    for (a, b) in _NET:
        va, vb, pa, pb = vals[a], vals[b], pws[a], pws[b]
        keep = (va > vb) | ((va == vb) & (pa < pb))
        vals[a] = jnp.where(keep, va, vb)
        vals[b] = jnp.where(keep, vb, va)
        pws[a] = jnp.where(keep, pa, pb)
        pws[b] = jnp.where(keep, pb, pa)
    c = jnp.zeros((bn, D), jnp.int32)
    for s in range(S):
        c = c + pws[s]
        idx8_ref[s] = c - 8
        nxt = vals[s + 1] if s + 1 < S else 0.0
        diff_ref[s] = vals[s] - nxt


def _ln_body(y_ref, lnw_ref, lnb_ref, a_ref, o_ref):
    y = y_ref[...]
    mean = jnp.mean(y, axis=(0, 2), keepdims=True)
    var = jnp.mean((y - mean) ** 2, axis=(0, 2), keepdims=True)
    yn = (y - mean) * lax.rsqrt(var + LN_EPS)
    yn = yn * lnw_ref[...][:, None, :] + lnb_ref[...][:, None, :]
    a = a_ref[0]
    o_ref[...] = jnp.transpose(jnp.where(yn >= 0, yn, a * yn), (1, 0, 2))


def _make_sc_gather(n):
    n_per_w = n // NUM_WORKERS
    n_chunks = n_per_w // CN
    nd = n * D
    cnd = CN * D
    chw = S * cnd
    ohw = H * CN * D
    out_row = H * D
    mesh = plsc.VectorSubcoreMesh(core_axis_name="c", subcore_axis_name="s")

    @functools.partial(
        pl.kernel,
        mesh=mesh,
        compiler_params=pltpu.CompilerParams(needs_layout_passes=False),
        out_type=jax.ShapeDtypeStruct((n * out_row,), jnp.float32),
        scratch_types=[
            pltpu.VMEM((1024 * H,), jnp.float32),
            pltpu.VMEM((2 * chw,), jnp.int32),
            pltpu.VMEM((2 * chw,), jnp.float32),
            pltpu.VMEM((2 * ohw,), jnp.float32),
            pltpu.SemaphoreType.DMA,
            pltpu.SemaphoreType.DMA,
            pltpu.SemaphoreType.DMA,
        ],
    )
    def sc_gather(idx_hbm, diff_hbm, fm_hbm, out_hbm,
                  fm_v, idx_v, diff_v, out_v, in_sem0, in_sem1, out_sem):
        wid = lax.axis_index("s") * 2 + lax.axis_index("c")
        n0 = wid * n_per_w
        in_sems = (in_sem0, in_sem1)
        pltpu.sync_copy(fm_hbm, fm_v)

        def issue_in(ci, b):
            base = (n0 + ci * CN) * D
            off = b * chw
            for s in range(S):
                pltpu.async_copy(
                    idx_hbm.at[pl.ds(s * nd + base, cnd)],
                    idx_v.at[pl.ds(off + s * cnd, cnd)],
                    in_sems[b])
                pltpu.async_copy(
                    diff_hbm.at[pl.ds(s * nd + base, cnd)],
                    diff_v.at[pl.ds(off + s * cnd, cnd)],
                    in_sems[b])

        issue_in(0, 0)

        def pair_body(pi, carry):
            for b in range(2):
                ci = pi * 2 + b
                @pl.when(ci + 1 < n_chunks)
                def _():
                    issue_in(ci + 1, 1 - b)

                pltpu.make_async_copy(
                    idx_hbm.at[pl.ds(0, chw)],
                    idx_v.at[pl.ds(b * chw, chw)], in_sems[b]).wait()
                pltpu.make_async_copy(
                    diff_hbm.at[pl.ds(0, chw)],
                    diff_v.at[pl.ds(b * chw, chw)], in_sems[b]).wait()

                @pl.when(pi > 0)
                def _():
                    pltpu.make_async_copy(
                        idx_hbm.at[pl.ds(0, ohw)],
                        out_v.at[pl.ds(b * ohw, ohw)], out_sem).wait()

                @plsc.parallel_loop(0, CN * D // 16, unroll=4)
                def _(gi):
                    col = gi * 16
                    accs = [None] * H
                    for s in range(S):
                        iv = idx_v[pl.ds(b * chw + s * cnd + col, 16)]
                        dv = diff_v[pl.ds(b * chw + s * cnd + col, 16)]
                        for h in range(H):
                            g = plsc.load_gather(fm_v, [iv + h])
                            t = dv * g
                            accs[h] = t if s == 0 else accs[h] + t
                    for h in range(H):
                        out_v[pl.ds(b * ohw + h * cnd + col, 16)] = accs[h]
                base = (n0 + ci * CN) * D
                for h in range(H):
                    pltpu.async_copy(
                        out_v.at[pl.ds(b * ohw + h * cnd, cnd)],
                        out_hbm.at[pl.ds(h * nd + base, cnd)],
                        out_sem)
            return carry

        lax.fori_loop(0, n_chunks // 2, pair_body, 0)
        for b in range(2):
            pltpu.make_async_copy(
                idx_hbm.at[pl.ds(0, ohw)],
                out_v.at[pl.ds(b * ohw, ohw)], out_sem).wait()

    return sc_gather


def kernel(x, FM, ln_w, ln_b, prelu_a):
    n = x.shape[0]
    x_t = jnp.transpose(x, (1, 0, 2))
    bn1 = 256
    idx8, diff = pl.pallas_call(
        _sortnet_body,
        grid=(n // bn1,),
        in_specs=[pl.BlockSpec((S, bn1, D), lambda i: (0, i, 0))],
        out_specs=[
            pl.BlockSpec((S, bn1, D), lambda i: (0, i, 0)),
            pl.BlockSpec((S, bn1, D), lambda i: (0, i, 0)),
        ],
        out_shape=[
            jax.ShapeDtypeStruct((S, n, D), jnp.int32),
            jax.ShapeDtypeStruct((S, n, D), jnp.float32),
        ],
    )(x_t)

    fm_flat = jnp.concatenate([FM.reshape(-1), jnp.zeros((8,), FM.dtype)])
    out = _make_sc_gather(n)(idx8.reshape(-1), diff.reshape(-1), fm_flat)
    out = out.reshape(H, n, D)

    bn3 = 512
    y = pl.pallas_call(
        _ln_body,
        grid=(n // bn3,),
        in_specs=[
            pl.BlockSpec((H, bn3, D), lambda i: (0, i, 0)),
            pl.BlockSpec((H, D), lambda i: (0, 0)),
            pl.BlockSpec((H, D), lambda i: (0, 0)),
            pl.BlockSpec((1,), lambda i: (0,)),
        ],
        out_specs=pl.BlockSpec((bn3, H, D), lambda i: (i, 0, 0)),
        out_shape=jax.ShapeDtypeStruct((n, H, D), jnp.float32),
    )(out, ln_w, ln_b, prelu_a)
    return y


# speedup vs baseline: 1.8411x; 1.8411x over previous
import functools

import jax
import jax.numpy as jnp
from jax import lax
from jax.experimental import pallas as pl
from jax.experimental.pallas import tpu as pltpu
from jax.experimental.pallas import tpu_sc as plsc

S = 10
H = 8
D = 128
LN_EPS = 1e-5

NUM_WORKERS = 32
CN = 8


def _oddeven_merge(lo, hi, r):
    step = r * 2
    if step < hi - lo:
        yield from _oddeven_merge(lo, hi, step)
        yield from _oddeven_merge(lo + r, hi, step)
        yield from ((i, i + r) for i in range(lo + r, hi - r, step))
    else:
        yield (lo, lo + r)


def _oddeven_sort(lo, hi):
    if (hi - lo) >= 1:
        mid = lo + ((hi - lo) // 2)
        yield from _oddeven_sort(lo, mid)
        yield from _oddeven_sort(mid + 1, hi)
        yield from _oddeven_merge(lo, hi + 1, 1)


def _network(n):
    p = 1
    while p < n:
        p *= 2
    return [(a, b) for (a, b) in _oddeven_sort(0, p - 1) if b < n]


_NET = _network(S)


def _sortnet_body(x_ref, idx8_ref, diff_ref):
    vals = [x_ref[i] for i in range(S)]
    bn = vals[0].shape[0]
    pws = [jnp.full((bn, D), 9 * (1 << i), jnp.int32) for i in range(S)]
    for (a, b) in _NET:
        va, vb, pa, pb = vals[a], vals[b], pws[a], pws[b]
        keep = (va > vb) | ((va == vb) & (pa < pb))
        vals[a] = jnp.where(keep, va, vb)
        vals[b] = jnp.where(keep, vb, va)
        pws[a] = jnp.where(keep, pa, pb)
        pws[b] = jnp.where(keep, pb, pa)
    c = jnp.zeros((bn, D), jnp.int32)
    for s in range(S):
        c = c + pws[s]
        if s < S - 1:
            idx8_ref[s] = c - 9
        nxt = vals[s + 1] if s + 1 < S else 0.0
        diff_ref[s] = vals[s] - nxt


def _ln_body(y_ref, lnw_ref, lnb_ref, a_ref, o_ref):
    y = y_ref[...]
    mean = jnp.mean(y, axis=(0, 2), keepdims=True)
    var = jnp.mean((y - mean) ** 2, axis=(0, 2), keepdims=True)
    yn = (y - mean) * lax.rsqrt(var + LN_EPS)
    yn = yn * lnw_ref[...][:, None, :] + lnb_ref[...][:, None, :]
    a = a_ref[0]
    o_ref[...] = jnp.transpose(jnp.where(yn >= 0, yn, a * yn), (1, 0, 2))


def _make_sc_gather(n):
    n_per_w = n // NUM_WORKERS
    n_chunks = n_per_w // CN
    nd = n * D
    cnd = CN * D
    chwi = (S - 1) * cnd
    chwd = S * cnd
    ohw = H * CN * D
    out_row = H * D
    mesh = plsc.VectorSubcoreMesh(core_axis_name="c", subcore_axis_name="s")

    @functools.partial(
        pl.kernel,
        mesh=mesh,
        compiler_params=pltpu.CompilerParams(needs_layout_passes=False),
        out_type=jax.ShapeDtypeStruct((n * out_row,), jnp.float32),
        scratch_types=[
            pltpu.VMEM((9216,), jnp.float32),
            pltpu.VMEM((2 * chwi,), jnp.int32),
            pltpu.VMEM((2 * chwd,), jnp.float32),
            pltpu.VMEM((2 * ohw,), jnp.float32),
            pltpu.SemaphoreType.DMA,
            pltpu.SemaphoreType.DMA,
            pltpu.SemaphoreType.DMA,
            pltpu.SemaphoreType.DMA,
        ],
    )
    def sc_gather(idx_hbm, diff_hbm, fm_hbm, out_hbm,
                  fm_v, idx_v, diff_v, out_v,
                  in_sem0, in_sem1, out_sem0, out_sem1):
        wid = lax.axis_index("s") * 2 + lax.axis_index("c")
        n0 = wid * n_per_w
        in_sems = (in_sem0, in_sem1)
        out_sems = (out_sem0, out_sem1)
        pltpu.sync_copy(fm_hbm, fm_v)
        g9 = [plsc.load_gather(fm_v, [jnp.full((16,), 1022 * 9 + h, jnp.int32)])
              for h in range(H)]

        def issue_in(ci, b):
            base = (n0 + ci * CN) * D
            for s in range(S - 1):
                pltpu.async_copy(
                    idx_hbm.at[pl.ds(s * nd + base, cnd)],
                    idx_v.at[pl.ds(b * chwi + s * cnd, cnd)],
                    in_sems[b])
            for s in range(S):
                pltpu.async_copy(
                    diff_hbm.at[pl.ds(s * nd + base, cnd)],
                    diff_v.at[pl.ds(b * chwd + s * cnd, cnd)],
                    in_sems[b])

        issue_in(0, 0)

        def pair_body(pi, carry):
            for b in range(2):
                ci = pi * 2 + b
                @pl.when(ci + 1 < n_chunks)
                def _():
                    issue_in(ci + 1, 1 - b)

                pltpu.make_async_copy(
                    idx_hbm.at[pl.ds(0, chwi)],
                    idx_v.at[pl.ds(b * chwi, chwi)], in_sems[b]).wait()
                pltpu.make_async_copy(
                    diff_hbm.at[pl.ds(0, chwd)],
                    diff_v.at[pl.ds(b * chwd, chwd)], in_sems[b]).wait()

                @pl.when(pi > 0)
                def _():
                    pltpu.make_async_copy(
                        idx_hbm.at[pl.ds(0, ohw)],
                        out_v.at[pl.ds(b * ohw, ohw)], out_sems[b]).wait()

                def g_body(gi, carry2):
                    for u in range(2):
                        col = gi * 32 + u * 16
                        accs = [None] * H
                        for s in range(S - 1):
                            iv = idx_v[pl.ds(b * chwi + s * cnd + col, 16)]
                            dv = diff_v[pl.ds(b * chwd + s * cnd + col, 16)]
                            for h in range(H):
                                g = plsc.load_gather(fm_v, [iv + h])
                                t = dv * g
                                accs[h] = t if s == 0 else accs[h] + t
                        dv9 = diff_v[pl.ds(b * chwd + (S - 1) * cnd + col, 16)]
                        for h in range(H):
                            accs[h] = accs[h] + dv9 * g9[h]
                        for h in range(H):
                            out_v[pl.ds(b * ohw + h * cnd + col, 16)] = accs[h]
                    return carry2

                lax.fori_loop(0, CN * D // 32, g_body, 0)
                base = (n0 + ci * CN) * D
                for h in range(H):
                    pltpu.async_copy(
                        out_v.at[pl.ds(b * ohw + h * cnd, cnd)],
                        out_hbm.at[pl.ds(h * nd + base, cnd)],
                        out_sems[b])
            return carry

        lax.fori_loop(0, n_chunks // 2, pair_body, 0)
        for b in range(2):
            pltpu.make_async_copy(
                idx_hbm.at[pl.ds(0, ohw)],
                out_v.at[pl.ds(b * ohw, ohw)], out_sems[b]).wait()

    return sc_gather


def kernel(x, FM, ln_w, ln_b, prelu_a):
    n = x.shape[0]
    x_t = jnp.transpose(x, (1, 0, 2))
    bn1 = 256
    idx8, diff = pl.pallas_call(
        _sortnet_body,
        grid=(n // bn1,),
        in_specs=[pl.BlockSpec((S, bn1, D), lambda i: (0, i, 0))],
        out_specs=[
            pl.BlockSpec((S - 1, bn1, D), lambda i: (0, i, 0)),
            pl.BlockSpec((S, bn1, D), lambda i: (0, i, 0)),
        ],
        out_shape=[
            jax.ShapeDtypeStruct((S - 1, n, D), jnp.int32),
            jax.ShapeDtypeStruct((S, n, D), jnp.float32),
        ],
    )(x_t)

    fm_pad = jnp.concatenate([FM, jnp.zeros((FM.shape[0], 1), FM.dtype)], axis=1)
    fm_flat = jnp.concatenate([fm_pad.reshape(-1), jnp.zeros((9,), FM.dtype)])
    out = _make_sc_gather(n)(idx8.reshape(-1), diff.reshape(-1), fm_flat)
    out = out.reshape(H, n, D)

    bn3 = 512
    y = pl.pallas_call(
        _ln_body,
        grid=(n // bn3,),
        in_specs=[
            pl.BlockSpec((H, bn3, D), lambda i: (0, i, 0)),
            pl.BlockSpec((H, D), lambda i: (0, 0)),
            pl.BlockSpec((H, D), lambda i: (0, 0)),
            pl.BlockSpec((1,), lambda i: (0,)),
        ],
        out_specs=pl.BlockSpec((bn3, H, D), lambda i: (i, 0, 0)),
        out_shape=jax.ShapeDtypeStruct((n, H, D), jnp.float32),
    )(out, ln_w, ln_b, prelu_a)
    return y
